# baseline (speedup 1.0000x reference)
import jax
import jax.numpy as jnp
from jax.experimental import pallas as pl
from jax.experimental.pallas import tpu as pltpu

_B = 16384
_P = 1024
_D = 128
_K = _D + 2
_BZ = 1024
_NB = _B // _BZ
_NC = 4
_BC = _BZ // _NC
_REG1 = 0.05
_REG2 = 0.05


def _loss_body(z_ref, p_ref, out_ref, paug_ref, colmin_ref, rowacc_ref):
    i = pl.program_id(0)

    @pl.when(i == 0)
    def _prep():
        p = p_ref[:]
        p2 = jnp.sum(p * p, axis=1, keepdims=True)
        paug_ref[:, :_D] = (-2.0 * p).astype(jnp.bfloat16)
        paug_ref[:, _D:_D + 1] = jnp.ones((_P, 1), jnp.bfloat16)
        paug_ref[:, _D + 1:] = p2.astype(jnp.bfloat16)

    rowparts = []
    colpart = None
    for c in range(_NC):
        zb = z_ref[pl.ds(c * _BC, _BC), :]
        z2 = jnp.sum(zb * zb, axis=1, keepdims=True)
        zaug = jnp.concatenate(
            [zb.astype(jnp.bfloat16), z2.astype(jnp.bfloat16),
             jnp.ones((_BC, 1), jnp.bfloat16)], axis=1)
        d2 = jax.lax.dot_general(
            zaug, paug_ref[:], (((1,), (1,)), ((), ())),
            preferred_element_type=jnp.float32)
        rowmin = jnp.min(d2, axis=1, keepdims=True)
        rowparts.append(jnp.sqrt(jnp.maximum(rowmin, 0.0)))
        cp = jnp.min(d2, axis=0, keepdims=True)
        colpart = cp if colpart is None else jnp.minimum(colpart, cp)
    rowpart = jnp.concatenate(rowparts, axis=0)

    @pl.when(i == 0)
    def _init():
        rowacc_ref[:] = rowpart
        colmin_ref[:] = colpart

    @pl.when(i > 0)
    def _accum():
        rowacc_ref[:] = rowacc_ref[:] + rowpart
        colmin_ref[:] = jnp.minimum(colmin_ref[:], colpart)

    @pl.when(i == _NB - 1)
    def _finish():
        cm = jnp.sqrt(jnp.maximum(colmin_ref[:], 0.0))
        val = (_REG1 * (jnp.sum(rowacc_ref[:]) / _B)
               + _REG2 * (jnp.sum(cm) / _P))
        out_ref[...] = jnp.reshape(val, (1, 1))


def kernel(z, prototype_vectors):
    out = pl.pallas_call(
        _loss_body,
        grid=(_NB,),
        in_specs=[
            pl.BlockSpec((_BZ, _D), lambda i: (i, 0)),
            pl.BlockSpec((_P, _D), lambda i: (0, 0)),
        ],
        out_specs=pl.BlockSpec((1, 1), lambda i: (0, 0)),
        out_shape=jax.ShapeDtypeStruct((1, 1), jnp.float32),
        scratch_shapes=[
            pltpu.VMEM((_P, _K), jnp.bfloat16),
            pltpu.VMEM((1, _P), jnp.float32),
            pltpu.VMEM((_BZ, 1), jnp.float32),
        ],
    )(z, prototype_vectors)
    return out[0, 0]


# speedup vs baseline: 1.3614x; 1.3614x over previous
import jax
import jax.numpy as jnp
from jax.experimental import pallas as pl
from jax.experimental.pallas import tpu as pltpu

_B = 16384
_P = 1024
_D = 128
_K = _D + 2
_BZ = 8192
_NB = _B // _BZ
_BC = 512
_NC = _BZ // _BC
_REG1 = 0.05
_REG2 = 0.05


def _loss_body(z_ref, p_ref, out_ref, paug_ref, colmin_ref, rowacc_ref):
    i = pl.program_id(0)

    @pl.when(i == 0)
    def _prep():
        p = p_ref[:]
        p2 = jnp.sum(p * p, axis=1, keepdims=True)
        paug_ref[:, :_D] = (-2.0 * p).astype(jnp.bfloat16)
        paug_ref[:, _D:_D + 1] = jnp.ones((_P, 1), jnp.bfloat16)
        paug_ref[:, _D + 1:] = p2.astype(jnp.bfloat16)

    rowacc = None
    colmin = None
    for c in range(_NC):
        zb = z_ref[pl.ds(c * _BC, _BC), :]
        z2 = jnp.sum(zb * zb, axis=1, keepdims=True)
        zaug = jnp.concatenate(
            [zb.astype(jnp.bfloat16), z2.astype(jnp.bfloat16),
             jnp.ones((_BC, 1), jnp.bfloat16)], axis=1)
        d2 = jax.lax.dot_general(
            zaug, paug_ref[:], (((1,), (1,)), ((), ())),
            preferred_element_type=jnp.float32).astype(jnp.bfloat16)
        rowmin = jnp.min(d2, axis=1, keepdims=True)
        rp = jnp.sqrt(jnp.maximum(rowmin.astype(jnp.float32), 0.0))
        rowacc = rp if rowacc is None else rowacc + rp
        cp = jnp.min(d2, axis=0, keepdims=True)
        colmin = cp if colmin is None else jnp.minimum(colmin, cp)

    @pl.when(i == 0)
    def _init():
        rowacc_ref[:] = rowacc
        colmin_ref[:] = colmin

    @pl.when(i > 0)
    def _accum():
        rowacc_ref[:] = rowacc_ref[:] + rowacc
        colmin_ref[:] = jnp.minimum(colmin_ref[:], colmin)

    @pl.when(i == _NB - 1)
    def _finish():
        cm = jnp.sqrt(jnp.maximum(
            colmin_ref[:].astype(jnp.float32), 0.0))
        val = (_REG1 * (jnp.sum(rowacc_ref[:]) / _B)
               + _REG2 * (jnp.sum(cm) / _P))
        out_ref[...] = jnp.reshape(val, (1, 1))


def kernel(z, prototype_vectors):
    out = pl.pallas_call(
        _loss_body,
        grid=(_NB,),
        in_specs=[
            pl.BlockSpec((_BZ, _D), lambda i: (i, 0)),
            pl.BlockSpec((_P, _D), lambda i: (0, 0)),
        ],
        out_specs=pl.BlockSpec((1, 1), lambda i: (0, 0)),
        out_shape=jax.ShapeDtypeStruct((1, 1), jnp.float32),
        scratch_shapes=[
            pltpu.VMEM((_P, _K), jnp.bfloat16),
            pltpu.VMEM((1, _P), jnp.bfloat16),
            pltpu.VMEM((_BC, 1), jnp.float32),
        ],
    )(z, prototype_vectors)
    return out[0, 0]
